# baseline (speedup 1.0000x reference)
import jax
import jax.numpy as jnp
import numpy as np
from jax.experimental import pallas as pl
from jax.experimental.pallas import tpu as pltpu
from jax.sharding import Mesh, PartitionSpec as P

_TILE = 8192


def _gates_kernel(x_ref, w_ref, b_ref, nh_ref, nc_ref, eh_ref, ec_ref):
    H = nh_ref.shape[-1]
    x = x_ref[...].astype(jnp.bfloat16)
    w = w_ref[...].astype(jnp.bfloat16)
    gates = jnp.dot(x, w, preferred_element_type=jnp.float32)
    gates = gates + b_ref[...]
    i = gates[:, :H]
    g = gates[:, 2 * H:3 * H]
    o = gates[:, 3 * H:]
    c = (0.5 + 0.5 * jnp.tanh(0.5 * i)) * jnp.tanh(g)
    h = (0.5 + 0.5 * jnp.tanh(0.5 * o)) * jnp.tanh(o)
    nh_ref[...] = h
    nc_ref[...] = c
    eh_ref[...] = h
    ec_ref[...] = c


def kernel(edge_input, W_ih, W_hh, bias):
    B, L, D = edge_input.shape
    H = W_hh.shape[0]
    x = edge_input.reshape(B * L, D)
    b = bias.reshape(1, 4 * H)
    n = B * L
    devs = jax.devices()
    ndev = 2 if len(devs) >= 2 and n % (2 * _TILE) == 0 else 1
    mesh = Mesh(np.array(devs[:ndev]), ("x",))
    n_loc = n // ndev
    tile = min(_TILE, n_loc)

    def run(xs, w, bb):
        out_spec = pl.BlockSpec((tile, H), lambda r: (r, 0))
        out_shape = jax.ShapeDtypeStruct((n_loc, H), edge_input.dtype)
        return pl.pallas_call(
            _gates_kernel,
            grid=(n_loc // tile,),
            in_specs=[
                pl.BlockSpec((tile, D), lambda r: (r, 0)),
                pl.BlockSpec((D, 4 * H), lambda r: (0, 0)),
                pl.BlockSpec((1, 4 * H), lambda r: (0, 0)),
            ],
            out_specs=[out_spec, out_spec, out_spec, out_spec],
            out_shape=[out_shape, out_shape, out_shape, out_shape],
            compiler_params=pltpu.CompilerParams(
                dimension_semantics=("parallel",),
            ),
        )(xs, w, bb)

    sharded = jax.shard_map(
        run, mesh=mesh,
        in_specs=(P("x", None), P(None, None), P(None, None)),
        out_specs=(P("x", None),) * 4,
        check_vma=False,
    )
    nh, nc, eh, ec = sharded(x, W_ih, b)
    return (nh.reshape(B, L, H), nc.reshape(B, L, H),
            eh.reshape(B, L, H), ec.reshape(B, L, H))


# speedup vs baseline: 16.1815x; 16.1815x over previous
import jax
import jax.numpy as jnp
from jax.experimental import pallas as pl
from jax.experimental.pallas import tpu as pltpu

_TILE = 8192


def _gates_kernel(x_ref, w_ref, b_ref, nh_ref, nc_ref, eh_ref, ec_ref):
    H = nh_ref.shape[-1]
    x = x_ref[...].astype(jnp.bfloat16)
    w = w_ref[...].astype(jnp.bfloat16)
    gates = jnp.dot(x, w, preferred_element_type=jnp.float32)
    gates = gates + b_ref[...]
    i = gates[:, :H]
    g = gates[:, 2 * H:3 * H]
    o = gates[:, 3 * H:]
    c = (0.5 + 0.5 * jnp.tanh(0.5 * i)) * jnp.tanh(g)
    h = (0.5 + 0.5 * jnp.tanh(0.5 * o)) * jnp.tanh(o)
    nh_ref[...] = h
    nc_ref[...] = c
    eh_ref[...] = h
    ec_ref[...] = c


def kernel(edge_input, W_ih, W_hh, bias):
    B, L, D = edge_input.shape
    H = W_hh.shape[0]
    x = edge_input.reshape(B * L, D)
    b = bias.reshape(1, 4 * H)
    n = B * L
    out_spec = pl.BlockSpec((_TILE, H), lambda r: (r, 0))
    out_shape = jax.ShapeDtypeStruct((n, H), edge_input.dtype)
    nh, nc, eh, ec = pl.pallas_call(
        _gates_kernel,
        grid=(n // _TILE,),
        in_specs=[
            pl.BlockSpec((_TILE, D), lambda r: (r, 0)),
            pl.BlockSpec((D, 4 * H), lambda r: (0, 0)),
            pl.BlockSpec((1, 4 * H), lambda r: (0, 0)),
        ],
        out_specs=[out_spec, out_spec, out_spec, out_spec],
        out_shape=[out_shape, out_shape, out_shape, out_shape],
        compiler_params=pltpu.CompilerParams(
            dimension_semantics=("parallel",),
        ),
    )(x, W_ih, b)
    return (nh.reshape(B, L, H), nc.reshape(B, L, H),
            eh.reshape(B, L, H), ec.reshape(B, L, H))
